# Initial kernel scaffold; baseline (speedup 1.0000x reference)
import jax
import jax.numpy as jnp
from jax.experimental import pallas as pl


def kernel(edge_feats, edge_weights, W, b):
    raise NotImplementedError("write your pallas kernel here")


# speedup vs baseline: 1.4658x; 1.4658x over previous
import jax
import jax.numpy as jnp
from jax.experimental import pallas as pl

NUM_NODES = 12
NUM_EDGES = 36
NODE_DIM = 128
TB = 256


def _fused_body(ew_ref, w_ref, b_ref, x_ref, out_ref, wts_ref):
    wts = jax.nn.softplus(ew_ref[:, :])
    wts_ref[:, :] = wts
    wmat = w_ref[:, :]
    bias = b_ref[:, :]
    for n in range(NUM_NODES):
        if n < 6:
            es = [6 * n + j for j in range(6)]
        else:
            es = [6 * i + (n - 6) for i in range(6)]
        acc = x_ref[:, es[0], :] * wts[0:1, es[0]:es[0] + 1]
        for e in es[1:]:
            acc = acc + x_ref[:, e, :] * wts[0:1, e:e + 1]
        y = jax.lax.dot_general(acc, wmat, (((1,), (1,)), ((), ())),
                                preferred_element_type=jnp.float32)
        out_ref[:, n, :] = jnp.maximum(y + bias, 0.0)


def kernel(edge_feats, edge_weights, W, b):
    B, E, D = edge_feats.shape
    ew2 = edge_weights.reshape(1, E)
    b2 = b.reshape(1, D)
    grid = (B // TB,)
    nodes, wts = pl.pallas_call(
        _fused_body,
        grid=grid,
        in_specs=[
            pl.BlockSpec((1, E), lambda i: (0, 0)),
            pl.BlockSpec((D, D), lambda i: (0, 0)),
            pl.BlockSpec((1, D), lambda i: (0, 0)),
            pl.BlockSpec((TB, E, D), lambda i: (i, 0, 0)),
        ],
        out_specs=[
            pl.BlockSpec((TB, NUM_NODES, D), lambda i: (i, 0, 0)),
            pl.BlockSpec((1, E), lambda i: (0, 0)),
        ],
        out_shape=[
            jax.ShapeDtypeStruct((B, NUM_NODES, D), edge_feats.dtype),
            jax.ShapeDtypeStruct((1, E), edge_weights.dtype),
        ],
    )(ew2, W, b2, edge_feats)
    return (nodes, wts.reshape(E))
